# Initial kernel scaffold; baseline (speedup 1.0000x reference)
import jax
import jax.numpy as jnp
from jax.experimental import pallas as pl


def kernel(positions, tables):
    raise NotImplementedError("write your pallas kernel here")


# speedup vs baseline: 29.1168x; 29.1168x over previous
import functools

import jax
import jax.numpy as jnp
from jax import lax
from jax.experimental import pallas as pl
from jax.experimental.pallas import tpu as pltpu
from jax.experimental.pallas import tpu_sc as plsc

_NUM_LEVELS = 8
_RESOLUTION = 32
_TABLE_SIZE = 32768
_FEATURE_DIM = 8
_N_POS = 786432
_OUT_DIM = _NUM_LEVELS * _FEATURE_DIM

_NUM_WORKERS = 32
_PER_WORKER = _N_POS // _NUM_WORKERS
_CHUNK = 1024
_NUM_CHUNKS = _PER_WORKER // _CHUNK
_GROWS = 128
_NUM_GATHERS = _CHUNK // _GROWS
_LANES = 16

_mesh = plsc.VectorSubcoreMesh(core_axis_name="c", subcore_axis_name="s")


@functools.partial(
    pl.kernel,
    mesh=_mesh,
    compiler_params=pltpu.CompilerParams(use_tc_tiling_on_sc=False),
    out_type=jax.ShapeDtypeStruct((_N_POS, _OUT_DIM), jnp.float32),
    scratch_types=[
        pltpu.VMEM((3, _CHUNK), jnp.float32),
        pltpu.VMEM((_NUM_GATHERS, _GROWS), jnp.int32),
        pltpu.VMEM((_CHUNK, _OUT_DIM), jnp.float32),
        pltpu.SemaphoreType.DMA,
    ],
)
def _encode(pos_hbm, table_hbm, out_hbm, pos_v, idx_v, rows_v, gsem):
    wid = lax.axis_index("s") * 2 + lax.axis_index("c")
    w_base = wid * _PER_WORKER

    def chunk_body(c, carry):
        base = w_base + c * _CHUNK
        pltpu.sync_copy(pos_hbm.at[:, pl.ds(base, _CHUNK)], pos_v)
        for g in range(_NUM_GATHERS):
            for v in range(_GROWS // _LANES):
                s = g * _GROWS + v * _LANES
                x = pos_v[0, pl.ds(s, _LANES)]
                y = pos_v[1, pl.ds(s, _LANES)]
                z = pos_v[2, pl.ds(s, _LANES)]
                fx = jnp.clip((x + 1.0) * 0.5 * _RESOLUTION, 0.0, _RESOLUTION - 1)
                fy = jnp.clip((y + 1.0) * 0.5 * _RESOLUTION, 0.0, _RESOLUTION - 1)
                fz = jnp.clip((z + 1.0) * 0.5 * _RESOLUTION, 0.0, _RESOLUTION - 1)
                f = fx * float(_RESOLUTION * _RESOLUTION) + fy * float(_RESOLUTION) + fz
                idx_v[g, pl.ds(v * _LANES, _LANES)] = f.astype(jnp.int32)
        copies = []
        for g in range(_NUM_GATHERS):
            cp = pltpu.make_async_copy(
                table_hbm.at[idx_v.at[g]],
                rows_v.at[pl.ds(g * _GROWS, _GROWS)],
                gsem,
            )
            cp.start()
            copies.append(cp)
        for cp in copies:
            cp.wait()
        pltpu.sync_copy(rows_v, out_hbm.at[pl.ds(base, _CHUNK)])
        return carry

    lax.fori_loop(0, _NUM_CHUNKS, chunk_body, 0)


def kernel(positions, tables):
    fused = jnp.transpose(tables, (1, 0, 2)).reshape(_TABLE_SIZE, _OUT_DIM)
    pos_t = jnp.transpose(positions)
    return _encode(pos_t, fused)
